# baseline (speedup 1.0000x reference)
import functools

import jax
import jax.numpy as jnp
from jax import lax
from jax.experimental import pallas as pl
from jax.experimental.pallas import tpu as pltpu
from jax.experimental.pallas import tpu_sc as plsc

B, N1, N2, C = 8, 4096, 1024, 128
EPS = 1e-5
BLK = 512
NB = N1 // BLK
NBLK = B * NB
CHQ = 128
NW = 32
BLK_PER_W = NBLK // NW
BLKA = 1024
NBA = N1 // BLKA
BLKB = 2048
NBB = N1 // BLKB
NTOT = B * N1


def _k1_body(x1_ref, x2_ref, gidx_ref, wq_ref):
    b = pl.program_id(0)
    d = None
    for c in range(3):
        t = x1_ref[0, c, :][:, None] - x2_ref[0, c, :][None, :]
        d = t * t if d is None else d + t * t
    it2 = lax.broadcasted_iota(jnp.int32, (N2, 2), 0)
    itcols = jnp.where(lax.broadcasted_iota(jnp.int32, (N2, 2), 1) == 0,
                       it2 >> 5, it2 & 31).astype(jnp.float32)
    inf = jnp.float32(jnp.inf)
    ws = []
    for k in range(3):
        m = jnp.min(d, axis=1, keepdims=True)
        onef = jnp.where(d == m, 1.0, 0.0)
        ikf = lax.dot_general(onef, itcols, (((1,), (0,)), ((), ())),
                              preferred_element_type=jnp.float32)
        iki = ikf.astype(jnp.int32)
        ik = jnp.minimum((iki[:, 0:1] << 5) + iki[:, 1:2], N2 - 1)
        gidx_ref[0, pl.ds(k * BLK, BLK), :] = ik + b * N2
        ws.append(1.0 / m)
        if k < 2:
            d = jnp.where(d == m, inf, d)
    s = (ws[0] + ws[1]) + ws[2]
    for k in range(3):
        wq_ref[0, pl.ds(k * BLK, BLK), :] = ws[k] / s


def _k1(xyz1t, xyz2t):
    return pl.pallas_call(
        _k1_body,
        grid=(B, NB),
        in_specs=[
            pl.BlockSpec((1, 3, BLK), lambda b, j: (b, 0, j)),
            pl.BlockSpec((1, 3, N2), lambda b, j: (b, 0, 0)),
        ],
        out_specs=[
            pl.BlockSpec((1, 3 * BLK, 1), lambda b, j: (b * NB + j, 0, 0)),
            pl.BlockSpec((1, 3 * BLK, 1), lambda b, j: (b * NB + j, 0, 0)),
        ],
        out_shape=[
            jax.ShapeDtypeStruct((NBLK, 3 * BLK, 1), jnp.int32),
            jax.ShapeDtypeStruct((NBLK, 3 * BLK, 1), jnp.float32),
        ],
    )(xyz1t, xyz2t)


def _sc_body(table_hbm, gidx_hbm, wq_hbm, fused_hbm,
             gidx_v, w_v, rows0, rows1, rows2, out_v, sem):
    wid = lax.axis_index("s") * 2 + lax.axis_index("c")
    rows = (rows0, rows1, rows2)
    for half in range(BLK_PER_W):
        blk = wid * BLK_PER_W + half
        pltpu.sync_copy(gidx_hbm.at[blk], gidx_v)
        pltpu.sync_copy(wq_hbm.at[blk], w_v)
        for t in range(BLK // CHQ):
            cps = [
                pltpu.async_copy(
                    table_hbm.at[gidx_v.at[pl.ds(k * BLK + t * CHQ, CHQ)]],
                    rows[k], sem)
                for k in range(3)
            ]
            for cp in cps:
                cp.wait()

            def body(g, _):
                base = t * CHQ + g * 16
                wv = [w_v[pl.ds(k * BLK + base, 16)] for k in range(3)]
                for i in range(16):
                    q = g * 16 + i
                    w0 = jnp.full((16,), wv[0][i], jnp.float32)
                    w1 = jnp.full((16,), wv[1][i], jnp.float32)
                    w2 = jnp.full((16,), wv[2][i], jnp.float32)
                    for cb in range(C // 16):
                        sl = pl.ds(cb * 16, 16)
                        out_v[q, sl] = (w0 * rows0[q, sl]
                                        + w1 * rows1[q, sl]) + w2 * rows2[q, sl]
                return 0

            lax.fori_loop(0, CHQ // 16, body, 0)
            pltpu.sync_copy(out_v, fused_hbm.at[blk, pl.ds(t * CHQ, CHQ)])


def _sc_gather(table, gidx, wq):
    kern = pl.kernel(
        _sc_body,
        out_type=jax.ShapeDtypeStruct((NBLK, BLK, C), jnp.float32),
        mesh=plsc.VectorSubcoreMesh(core_axis_name="c", subcore_axis_name="s",
                                    num_cores=2, num_subcores=16),
        scratch_types=[
            pltpu.VMEM((3 * BLK,), jnp.int32),
            pltpu.VMEM((3 * BLK,), jnp.float32),
            pltpu.VMEM((CHQ, C), jnp.float32),
            pltpu.VMEM((CHQ, C), jnp.float32),
            pltpu.VMEM((CHQ, C), jnp.float32),
            pltpu.VMEM((CHQ, C), jnp.float32),
            pltpu.SemaphoreType.DMA,
        ],
    )
    return kern(table, gidx.reshape(NBLK, 3 * BLK), wq.reshape(NBLK, 3 * BLK))


def _accum_stats(h, s_ref, q_ref):
    s_blk = jnp.sum(h, axis=1, keepdims=True)
    q_blk = jnp.sum(h * h, axis=1, keepdims=True)
    first = (pl.program_id(0) == 0) & (pl.program_id(1) == 0)

    @pl.when(first)
    def _():
        s_ref[...] = s_blk
        q_ref[...] = q_blk

    @pl.when(jnp.logical_not(first))
    def _():
        s_ref[...] = s_ref[...] + s_blk
        q_ref[...] = q_ref[...] + q_blk


def _bn_affine(s_ref, q_ref, g_ref, be_ref):
    mean = s_ref[...] * (1.0 / NTOT)
    var = q_ref[...] * (1.0 / NTOT) - mean * mean
    a = g_ref[...] / jnp.sqrt(var + EPS)
    c = be_ref[...] - mean * a
    return a, c


def _k2_body(p1_ref, f_ref, pb_ref, w_ref, b_ref, h_ref, s_ref, q_ref):
    fused = jnp.concatenate(
        [f_ref[u] for u in range(BLKA // BLK)], axis=0)
    w = w_ref[...]
    h = lax.dot_general(w[:, :C], p1_ref[0], (((1,), (0,)), ((), ())),
                        preferred_element_type=jnp.float32)
    h = h + lax.dot_general(w[:, C:2 * C], fused, (((1,), (1,)), ((), ())),
                            preferred_element_type=jnp.float32)
    h = h + lax.dot_general(w[:, 2 * C:], pb_ref[0], (((1,), (0,)), ((), ())),
                            preferred_element_type=jnp.float32)
    h = h + b_ref[...]
    h_ref[0] = h.astype(jnp.bfloat16)
    _accum_stats(h, s_ref, q_ref)


def _k2(points1, fused, points_b1, W0, b0c):
    co = W0.shape[0]
    bpa = BLKA // BLK
    return pl.pallas_call(
        _k2_body,
        grid=(B, NBA),
        in_specs=[
            pl.BlockSpec((1, C, BLKA), lambda b, j: (b, 0, j)),
            pl.BlockSpec((bpa, BLK, C), lambda b, j: (b * NBA + j, 0, 0)),
            pl.BlockSpec((1, C, BLKA), lambda b, j: (b, 0, j)),
            pl.BlockSpec((co, 3 * C), lambda b, j: (0, 0)),
            pl.BlockSpec((co, 1), lambda b, j: (0, 0)),
        ],
        out_specs=[
            pl.BlockSpec((1, co, BLKA), lambda b, j: (b, 0, j)),
            pl.BlockSpec((co, 1), lambda b, j: (0, 0)),
            pl.BlockSpec((co, 1), lambda b, j: (0, 0)),
        ],
        out_shape=[
            jax.ShapeDtypeStruct((B, co, N1), jnp.bfloat16),
            jax.ShapeDtypeStruct((co, 1), jnp.float32),
            jax.ShapeDtypeStruct((co, 1), jnp.float32),
        ],
    )(points1, fused, points_b1, W0, b0c)


def _k3_body(h0_ref, s0_ref, q0_ref, g0_ref, be0_ref, w_ref, b_ref,
             h_ref, s_ref, q_ref):
    a, c = _bn_affine(s0_ref, q0_ref, g0_ref, be0_ref)
    xh = jnp.maximum(h0_ref[0].astype(jnp.float32) * a + c, 0.0)
    h = lax.dot_general(w_ref[...], xh, (((1,), (0,)), ((), ())),
                        preferred_element_type=jnp.float32)
    h = h + b_ref[...]
    h_ref[0] = h.astype(jnp.bfloat16)
    _accum_stats(h, s_ref, q_ref)


def _k3(h0, s0, q0, g0c, be0c, W1, b1c):
    ci, co = W1.shape[1], W1.shape[0]
    return pl.pallas_call(
        _k3_body,
        grid=(B, NBA),
        in_specs=[
            pl.BlockSpec((1, ci, BLKA), lambda b, j: (b, 0, j)),
            pl.BlockSpec((ci, 1), lambda b, j: (0, 0)),
            pl.BlockSpec((ci, 1), lambda b, j: (0, 0)),
            pl.BlockSpec((ci, 1), lambda b, j: (0, 0)),
            pl.BlockSpec((ci, 1), lambda b, j: (0, 0)),
            pl.BlockSpec((co, ci), lambda b, j: (0, 0)),
            pl.BlockSpec((co, 1), lambda b, j: (0, 0)),
        ],
        out_specs=[
            pl.BlockSpec((1, co, BLKA), lambda b, j: (b, 0, j)),
            pl.BlockSpec((co, 1), lambda b, j: (0, 0)),
            pl.BlockSpec((co, 1), lambda b, j: (0, 0)),
        ],
        out_shape=[
            jax.ShapeDtypeStruct((B, co, N1), jnp.bfloat16),
            jax.ShapeDtypeStruct((co, 1), jnp.float32),
            jax.ShapeDtypeStruct((co, 1), jnp.float32),
        ],
    )(h0, s0, q0, g0c, be0c, W1, b1c)


def _k4_body(h1_ref, s1_ref, q1_ref, g1_ref, be1_ref, o_ref):
    a, c = _bn_affine(s1_ref, q1_ref, g1_ref, be1_ref)
    y = h1_ref[0].astype(jnp.float32) * a + c
    o_ref[0, 0, :] = jnp.maximum(jnp.max(y, axis=0), 0.0)


def _k4(h1, s1, q1, g1c, be1c):
    ci = h1.shape[1]
    return pl.pallas_call(
        _k4_body,
        grid=(B, NBB),
        in_specs=[
            pl.BlockSpec((1, ci, BLKB), lambda b, j: (b, 0, j)),
            pl.BlockSpec((ci, 1), lambda b, j: (0, 0)),
            pl.BlockSpec((ci, 1), lambda b, j: (0, 0)),
            pl.BlockSpec((ci, 1), lambda b, j: (0, 0)),
            pl.BlockSpec((ci, 1), lambda b, j: (0, 0)),
        ],
        out_specs=pl.BlockSpec((1, 1, BLKB), lambda b, j: (b, 0, j)),
        out_shape=jax.ShapeDtypeStruct((B, 1, N1), jnp.float32),
    )(h1, s1, q1, g1c, be1c)


def kernel(xyz1, xyz2, points2, points1, points_b1,
           W0, b0, gamma0, beta0, W1, b1, gamma1, beta1):
    xyz1t = jnp.transpose(xyz1, (0, 2, 1))
    xyz2t = jnp.transpose(xyz2, (0, 2, 1))
    table = jnp.transpose(points2, (0, 2, 1)).reshape(B * N2, C)

    gidx, wq = _k1(xyz1t, xyz2t)
    fused = _sc_gather(table, gidx, wq)

    h0, s0, q0 = _k2(points1, fused, points_b1, W0, b0[:, None])
    h1, s1, q1 = _k3(h0, s0, q0, gamma0[:, None], beta0[:, None],
                     W1, b1[:, None])
    out = _k4(h1, s1, q1, gamma1[:, None], beta1[:, None])
    return out.reshape(B, N1)


# speedup vs baseline: 1.0684x; 1.0684x over previous
import functools

import jax
import jax.numpy as jnp
from jax import lax
from jax.experimental import pallas as pl
from jax.experimental.pallas import tpu as pltpu
from jax.experimental.pallas import tpu_sc as plsc

B, N1, N2, C = 8, 4096, 1024, 128
EPS = 1e-5
BLK = 512
NB = N1 // BLK
NBLK = B * NB
CHQ = 128
NW = 32
BLK_PER_W = NBLK // NW
BLKA = 1024
NBA = N1 // BLKA
BLKB = 2048
NBB = N1 // BLKB
NTOT = B * N1


def _k1_body(x1_ref, x2_ref, gidx_ref, wq_ref):
    b = pl.program_id(0)
    d = None
    for c in range(3):
        t = x1_ref[0, c, :][:, None] - x2_ref[0, c, :][None, :]
        d = t * t if d is None else d + t * t
    it2 = lax.broadcasted_iota(jnp.int32, (N2, 2), 0)
    itcols = jnp.where(lax.broadcasted_iota(jnp.int32, (N2, 2), 1) == 0,
                       it2 >> 5, it2 & 31).astype(jnp.float32)
    inf = jnp.float32(jnp.inf)
    ws = []
    for k in range(3):
        m = jnp.min(d, axis=1, keepdims=True)
        onef = jnp.where(d == m, 1.0, 0.0)
        ikf = lax.dot_general(onef, itcols, (((1,), (0,)), ((), ())),
                              preferred_element_type=jnp.float32)
        iki = ikf.astype(jnp.int32)
        ik = jnp.minimum((iki[:, 0:1] << 5) + iki[:, 1:2], N2 - 1)
        gidx_ref[0, pl.ds(k * BLK, BLK), :] = ik + b * N2
        ws.append(1.0 / m)
        if k < 2:
            d = jnp.where(d == m, inf, d)
    s = (ws[0] + ws[1]) + ws[2]
    for k in range(3):
        wq_ref[0, pl.ds(k * BLK, BLK), :] = ws[k] / s


def _k1(xyz1t, xyz2t):
    return pl.pallas_call(
        _k1_body,
        grid=(B, NB),
        in_specs=[
            pl.BlockSpec((1, 3, BLK), lambda b, j: (b, 0, j)),
            pl.BlockSpec((1, 3, N2), lambda b, j: (b, 0, 0)),
        ],
        out_specs=[
            pl.BlockSpec((1, 3 * BLK, 1), lambda b, j: (b * NB + j, 0, 0)),
            pl.BlockSpec((1, 3 * BLK, 1), lambda b, j: (b * NB + j, 0, 0)),
        ],
        out_shape=[
            jax.ShapeDtypeStruct((NBLK, 3 * BLK, 1), jnp.int32),
            jax.ShapeDtypeStruct((NBLK, 3 * BLK, 1), jnp.float32),
        ],
    )(xyz1t, xyz2t)


def _sc_body(table_hbm, gidx_hbm, gath_hbm, gidx_v, rows0, rows1, rows2, sem):
    wid = lax.axis_index("s") * 2 + lax.axis_index("c")
    rows = (rows0, rows1, rows2)
    for half in range(BLK_PER_W):
        blk = wid * BLK_PER_W + half
        pltpu.sync_copy(gidx_hbm.at[blk], gidx_v)
        for t in range(BLK // CHQ):
            cps = [
                pltpu.async_copy(
                    table_hbm.at[gidx_v.at[pl.ds(k * BLK + t * CHQ, CHQ)]],
                    rows[k], sem)
                for k in range(3)
            ]
            for cp in cps:
                cp.wait()
            for k in range(3):
                pltpu.sync_copy(
                    rows[k], gath_hbm.at[blk, k, pl.ds(t * CHQ, CHQ)])


def _sc_gather(table, gidx):
    kern = pl.kernel(
        _sc_body,
        out_type=jax.ShapeDtypeStruct((NBLK, 3, BLK, C), jnp.float32),
        mesh=plsc.VectorSubcoreMesh(core_axis_name="c", subcore_axis_name="s",
                                    num_cores=2, num_subcores=16),
        scratch_types=[
            pltpu.VMEM((3 * BLK,), jnp.int32),
            pltpu.VMEM((CHQ, C), jnp.float32),
            pltpu.VMEM((CHQ, C), jnp.float32),
            pltpu.VMEM((CHQ, C), jnp.float32),
            pltpu.SemaphoreType.DMA,
        ],
    )
    return kern(table, gidx.reshape(NBLK, 3 * BLK))


def _accum_stats(h, s_ref, q_ref):
    s_blk = jnp.sum(h, axis=1, keepdims=True)
    q_blk = jnp.sum(h * h, axis=1, keepdims=True)
    first = (pl.program_id(0) == 0) & (pl.program_id(1) == 0)

    @pl.when(first)
    def _():
        s_ref[...] = s_blk
        q_ref[...] = q_blk

    @pl.when(jnp.logical_not(first))
    def _():
        s_ref[...] = s_ref[...] + s_blk
        q_ref[...] = q_ref[...] + q_blk


def _bn_affine(s_ref, q_ref, g_ref, be_ref):
    mean = s_ref[...] * (1.0 / NTOT)
    var = q_ref[...] * (1.0 / NTOT) - mean * mean
    a = g_ref[...] / jnp.sqrt(var + EPS)
    c = be_ref[...] - mean * a
    return a, c


def _k2_body(p1_ref, g_ref, wq_ref, pb_ref, w_ref, b_ref, h_ref, s_ref, q_ref):
    fs = []
    for u in range(BLKA // BLK):
        wk = [wq_ref[u, pl.ds(k * BLK, BLK), :] for k in range(3)]
        fs.append((wk[0] * g_ref[u, 0] + wk[1] * g_ref[u, 1])
                  + wk[2] * g_ref[u, 2])
    fused = jnp.concatenate(fs, axis=0)
    w = w_ref[...]
    h = lax.dot_general(w[:, :C], p1_ref[0], (((1,), (0,)), ((), ())),
                        preferred_element_type=jnp.float32)
    h = h + lax.dot_general(w[:, C:2 * C], fused, (((1,), (1,)), ((), ())),
                            preferred_element_type=jnp.float32)
    h = h + lax.dot_general(w[:, 2 * C:], pb_ref[0], (((1,), (0,)), ((), ())),
                            preferred_element_type=jnp.float32)
    h = h + b_ref[...]
    h_ref[0] = h.astype(jnp.bfloat16)
    _accum_stats(h, s_ref, q_ref)


def _k2(points1, gath, wq, points_b1, W0, b0c):
    co = W0.shape[0]
    bpa = BLKA // BLK
    return pl.pallas_call(
        _k2_body,
        grid=(B, NBA),
        in_specs=[
            pl.BlockSpec((1, C, BLKA), lambda b, j: (b, 0, j)),
            pl.BlockSpec((bpa, 3, BLK, C), lambda b, j: (b * NBA + j, 0, 0, 0)),
            pl.BlockSpec((bpa, 3 * BLK, 1), lambda b, j: (b * NBA + j, 0, 0)),
            pl.BlockSpec((1, C, BLKA), lambda b, j: (b, 0, j)),
            pl.BlockSpec((co, 3 * C), lambda b, j: (0, 0)),
            pl.BlockSpec((co, 1), lambda b, j: (0, 0)),
        ],
        out_specs=[
            pl.BlockSpec((1, co, BLKA), lambda b, j: (b, 0, j)),
            pl.BlockSpec((co, 1), lambda b, j: (0, 0)),
            pl.BlockSpec((co, 1), lambda b, j: (0, 0)),
        ],
        out_shape=[
            jax.ShapeDtypeStruct((B, co, N1), jnp.bfloat16),
            jax.ShapeDtypeStruct((co, 1), jnp.float32),
            jax.ShapeDtypeStruct((co, 1), jnp.float32),
        ],
    )(points1, gath, wq, points_b1, W0, b0c)


def _k3_body(h0_ref, s0_ref, q0_ref, g0_ref, be0_ref, w_ref, b_ref,
             h_ref, s_ref, q_ref):
    a, c = _bn_affine(s0_ref, q0_ref, g0_ref, be0_ref)
    xh = jnp.maximum(h0_ref[0].astype(jnp.float32) * a + c, 0.0)
    h = lax.dot_general(w_ref[...], xh, (((1,), (0,)), ((), ())),
                        preferred_element_type=jnp.float32)
    h = h + b_ref[...]
    h_ref[0] = h.astype(jnp.bfloat16)
    _accum_stats(h, s_ref, q_ref)


def _k3(h0, s0, q0, g0c, be0c, W1, b1c):
    ci, co = W1.shape[1], W1.shape[0]
    return pl.pallas_call(
        _k3_body,
        grid=(B, NBA),
        in_specs=[
            pl.BlockSpec((1, ci, BLKA), lambda b, j: (b, 0, j)),
            pl.BlockSpec((ci, 1), lambda b, j: (0, 0)),
            pl.BlockSpec((ci, 1), lambda b, j: (0, 0)),
            pl.BlockSpec((ci, 1), lambda b, j: (0, 0)),
            pl.BlockSpec((ci, 1), lambda b, j: (0, 0)),
            pl.BlockSpec((co, ci), lambda b, j: (0, 0)),
            pl.BlockSpec((co, 1), lambda b, j: (0, 0)),
        ],
        out_specs=[
            pl.BlockSpec((1, co, BLKA), lambda b, j: (b, 0, j)),
            pl.BlockSpec((co, 1), lambda b, j: (0, 0)),
            pl.BlockSpec((co, 1), lambda b, j: (0, 0)),
        ],
        out_shape=[
            jax.ShapeDtypeStruct((B, co, N1), jnp.bfloat16),
            jax.ShapeDtypeStruct((co, 1), jnp.float32),
            jax.ShapeDtypeStruct((co, 1), jnp.float32),
        ],
    )(h0, s0, q0, g0c, be0c, W1, b1c)


def _k4_body(h1_ref, s1_ref, q1_ref, g1_ref, be1_ref, o_ref):
    a, c = _bn_affine(s1_ref, q1_ref, g1_ref, be1_ref)
    y = h1_ref[0].astype(jnp.float32) * a + c
    o_ref[0, 0, :] = jnp.maximum(jnp.max(y, axis=0), 0.0)


def _k4(h1, s1, q1, g1c, be1c):
    ci = h1.shape[1]
    return pl.pallas_call(
        _k4_body,
        grid=(B, NBB),
        in_specs=[
            pl.BlockSpec((1, ci, BLKB), lambda b, j: (b, 0, j)),
            pl.BlockSpec((ci, 1), lambda b, j: (0, 0)),
            pl.BlockSpec((ci, 1), lambda b, j: (0, 0)),
            pl.BlockSpec((ci, 1), lambda b, j: (0, 0)),
            pl.BlockSpec((ci, 1), lambda b, j: (0, 0)),
        ],
        out_specs=pl.BlockSpec((1, 1, BLKB), lambda b, j: (b, 0, j)),
        out_shape=jax.ShapeDtypeStruct((B, 1, N1), jnp.float32),
    )(h1, s1, q1, g1c, be1c)


def kernel(xyz1, xyz2, points2, points1, points_b1,
           W0, b0, gamma0, beta0, W1, b1, gamma1, beta1):
    xyz1t = jnp.transpose(xyz1, (0, 2, 1))
    xyz2t = jnp.transpose(xyz2, (0, 2, 1))
    table = jnp.transpose(points2, (0, 2, 1)).reshape(B * N2, C)

    gidx, wq = _k1(xyz1t, xyz2t)
    gath = _sc_gather(table, gidx)

    h0, s0, q0 = _k2(points1, gath, wq, points_b1, W0, b0[:, None])
    h1, s1, q1 = _k3(h0, s0, q0, gamma0[:, None], beta0[:, None],
                     W1, b1[:, None])
    out = _k4(h1, s1, q1, gamma1[:, None], beta1[:, None])
    return out.reshape(B, N1)
